# baseline (speedup 1.0000x reference)
import functools

import jax
import jax.numpy as jnp
from jax import lax
from jax.experimental import pallas as pl
from jax.experimental.pallas import tpu as pltpu
from jax.experimental.pallas import tpu_sc as plsc

_CUTOFF = 5.0
_LOG2 = 0.6931471805599453


def _sc_gather(table, idx):
    M, F = table.shape
    (R,) = idx.shape
    info = plsc.get_sparse_core_info()
    nw = info.num_cores * info.num_subcores
    per_w = R // nw
    RING = 5
    LOOK = 3
    CH = 8
    for c in range(128, 7, -8):
        if per_w % (c * RING) == 0:
            CH = c
            break
    assert per_w * nw == R and per_w % (CH * RING) == 0
    n_chunks = per_w // CH
    n_pass = n_chunks // RING
    mesh = plsc.VectorSubcoreMesh(core_axis_name="c", subcore_axis_name="s")

    @functools.partial(
        pl.kernel,
        mesh=mesh,
        out_type=jax.ShapeDtypeStruct((R, F), table.dtype),
        scratch_types=(
            [pltpu.VMEM((per_w,), jnp.int32)]
            + [pltpu.VMEM((CH, F), table.dtype)] * RING
            + [pltpu.SemaphoreType.DMA] * (2 * RING)
        ),
    )
    def k(table_hbm, idx_hbm, out_hbm, idx_v, *rest):
        bufs = rest[:RING]
        semg = rest[RING:2 * RING]
        semw = rest[2 * RING:]
        wid = lax.axis_index("s") * info.num_cores + lax.axis_index("c")
        base = wid * per_w
        pltpu.sync_copy(idx_hbm.at[pl.ds(base, per_w)], idx_v)

        def g_start(p, b):
            pltpu.make_async_copy(
                table_hbm.at[idx_v.at[pl.ds(p * CH, CH)]], bufs[b], semg[b]
            ).start()

        def g_wait(b):
            pltpu.make_async_copy(
                table_hbm.at[idx_v.at[pl.ds(0, CH)]], bufs[b], semg[b]
            ).wait()

        def w_start(p, b):
            pltpu.make_async_copy(
                bufs[b], out_hbm.at[pl.ds(base + p * CH, CH)], semw[b]
            ).start()

        def w_wait(b):
            pltpu.make_async_copy(
                bufs[b], out_hbm.at[pl.ds(base, CH)], semw[b]
            ).wait()

        def slot(p, b, wait_w, issue):
            g_wait(b)
            w_start(p, b)
            if issue:
                c = (b + LOOK) % RING
                if wait_w:
                    w_wait(c)
                g_start(p + LOOK, c)

        for b in range(LOOK):
            g_start(b, b)
        for b in range(RING):
            slot(b, b, wait_w=(b >= RING - LOOK), issue=True)

        def body(t, carry):
            p0 = t * RING
            for b in range(RING):
                slot(p0 + b, b, wait_w=True, issue=True)
            return carry

        lax.fori_loop(1, n_pass - 1, body, 0)

        p0 = (n_pass - 1) * RING
        for b in range(RING):
            slot(p0 + b, b, wait_w=True, issue=(p0 + b + LOOK < n_chunks))
        for b in range(RING):
            w_wait(b)

    return k(table, idx)


def _pick_tile(n, cap):
    best = 1
    best8 = 0
    for t in range(1, cap + 1):
        if n % t == 0:
            best = t
            if t % 8 == 0:
                best8 = t
    return best8 if best8 else best


def _in2f_kernel(x_ref, win_ref, y_ref):
    y_ref[...] = jnp.dot(x_ref[...], win_ref[...],
                         preferred_element_type=jnp.float32)


def _interaction_kernel(c_ref, r_ref, m_ref, ynb_ref,
                        wc_ref, wout_ref, bout_ref, wd_ref, bd_ref,
                        out_ref, *, T, N):
    F = wc_ref.shape[1]
    w = jnp.dot(c_ref[0], wc_ref[...], preferred_element_type=jnp.float32)
    r = r_ref[0]
    c = 0.5 * (jnp.cos(r * (jnp.pi / _CUTOFF)) + 1.0)
    c = c * (r < _CUTOFF).astype(jnp.float32) * m_ref[0]
    w = w.reshape(T, N, F) * c[:, :, None]
    y = ynb_ref[...].astype(jnp.float32).reshape(T, N, F)
    agg = jnp.sum(w * y, axis=1)
    v = jnp.dot(agg, wout_ref[...], preferred_element_type=jnp.float32)
    v = jax.nn.softplus(v + bout_ref[...]) - _LOG2
    out_ref[...] = jnp.dot(v, wd_ref[...],
                           preferred_element_type=jnp.float32) + bd_ref[...]


def kernel(x, r_ij, neighbors, neighbor_mask, fsblock_ij, fpblock_ij,
           Wfs, Wfp, Win, Wout, bout, Wd, bd):
    B, A, D = x.shape
    N = neighbors.shape[-1]
    S, F = Wfs.shape

    xf = x.reshape(B * A, D)
    Tk = _pick_tile(B * A, 2048)
    y = pl.pallas_call(
        _in2f_kernel,
        grid=(B * A // Tk,),
        in_specs=[
            pl.BlockSpec((Tk, D), lambda i: (i, 0)),
            pl.BlockSpec((D, F), lambda i: (0, 0)),
        ],
        out_specs=pl.BlockSpec((Tk, F), lambda i: (i, 0)),
        out_shape=jax.ShapeDtypeStruct((B * A, F), jnp.float32),
    )(xf, Win)

    cmb = jnp.concatenate([fsblock_ij, fpblock_ij],
                          axis=-1).astype(jnp.bfloat16).reshape(B, A * N, 2 * S)
    wc = jnp.concatenate([Wfs, Wfp], axis=0).astype(jnp.bfloat16)

    idx = (neighbors.reshape(B, A * N)
           + (jnp.arange(B, dtype=jnp.int32) * A)[:, None]).reshape(B * A * N)
    y_nbh = _sc_gather(y, idx)

    T = _pick_tile(A, 256)
    nt = A // T
    spec_edge = pl.BlockSpec((1, T * N, 2 * S), lambda b, i: (b, i, 0))
    spec_ynb = pl.BlockSpec((T * N, F), lambda b, i: (b * nt + i, 0))
    spec_an = pl.BlockSpec((1, T, N), lambda b, i: (b, i, 0))
    full = lambda shape: pl.BlockSpec(shape, lambda b, i: (0,) * len(shape))
    out = pl.pallas_call(
        functools.partial(_interaction_kernel, T=T, N=N),
        grid=(B, nt),
        in_specs=[
            spec_edge, spec_an, spec_an, spec_ynb,
            full((2 * S, F)), full((F, D)), full((1, D)),
            full((D, D)), full((1, D)),
        ],
        out_specs=pl.BlockSpec((T, D), lambda b, i: (b * nt + i, 0)),
        out_shape=jax.ShapeDtypeStruct((B * A, D), jnp.float32),
    )(cmb,
      r_ij, neighbor_mask, y_nbh,
      wc, Wout, bout.reshape(1, D), Wd, bd.reshape(1, D))
    return out.reshape(B, A, D)


# speedup vs baseline: 1.1344x; 1.1344x over previous
import functools

import jax
import jax.numpy as jnp
from jax import lax
from jax.experimental import pallas as pl
from jax.experimental.pallas import tpu as pltpu
from jax.experimental.pallas import tpu_sc as plsc

_CUTOFF = 5.0
_LOG2 = 0.6931471805599453


def _sc_gather(table, idx):
    M, F = table.shape
    (R,) = idx.shape
    info = plsc.get_sparse_core_info()
    nw = info.num_cores * info.num_subcores
    per_w = R // nw
    RING = 5
    LOOK = 3
    CH = 8
    for c in range(128, 7, -8):
        if per_w % (c * RING) == 0:
            CH = c
            break
    assert per_w * nw == R and per_w % (CH * RING) == 0
    n_chunks = per_w // CH
    n_pass = n_chunks // RING
    mesh = plsc.VectorSubcoreMesh(core_axis_name="c", subcore_axis_name="s")

    @functools.partial(
        pl.kernel,
        mesh=mesh,
        out_type=jax.ShapeDtypeStruct((R, F), table.dtype),
        scratch_types=(
            [pltpu.VMEM((per_w,), jnp.int32)]
            + [pltpu.VMEM((CH, F), table.dtype)] * RING
            + [pltpu.SemaphoreType.DMA] * (2 * RING)
        ),
    )
    def k(table_hbm, idx_hbm, out_hbm, idx_v, *rest):
        bufs = rest[:RING]
        semg = rest[RING:2 * RING]
        semw = rest[2 * RING:]
        wid = lax.axis_index("s") * info.num_cores + lax.axis_index("c")
        base = wid * per_w
        pltpu.sync_copy(idx_hbm.at[pl.ds(base, per_w)], idx_v)

        def g_start(p, b):
            pltpu.make_async_copy(
                table_hbm.at[idx_v.at[pl.ds(p * CH, CH)]], bufs[b], semg[b]
            ).start()

        def g_wait(b):
            pltpu.make_async_copy(
                table_hbm.at[idx_v.at[pl.ds(0, CH)]], bufs[b], semg[b]
            ).wait()

        def w_start(p, b):
            pltpu.make_async_copy(
                bufs[b], out_hbm.at[pl.ds(base + p * CH, CH)], semw[b]
            ).start()

        def w_wait(b):
            pltpu.make_async_copy(
                bufs[b], out_hbm.at[pl.ds(base, CH)], semw[b]
            ).wait()

        def slot(p, b, wait_w, issue):
            g_wait(b)
            w_start(p, b)
            if issue:
                c = (b + LOOK) % RING
                if wait_w:
                    w_wait(c)
                g_start(p + LOOK, c)

        for b in range(LOOK):
            g_start(b, b)
        for b in range(RING):
            slot(b, b, wait_w=(b >= RING - LOOK), issue=True)

        def body(t, carry):
            p0 = t * RING
            for b in range(RING):
                slot(p0 + b, b, wait_w=True, issue=True)
            return carry

        lax.fori_loop(1, n_pass - 1, body, 0)

        p0 = (n_pass - 1) * RING
        for b in range(RING):
            slot(p0 + b, b, wait_w=True, issue=(p0 + b + LOOK < n_chunks))
        for b in range(RING):
            w_wait(b)

    return k(table, idx)


def _pick_tile(n, cap):
    best = 1
    best8 = 0
    for t in range(1, cap + 1):
        if n % t == 0:
            best = t
            if t % 8 == 0:
                best8 = t
    return best8 if best8 else best


def _in2f_kernel(x_ref, win_ref, y_ref):
    y_ref[...] = jnp.dot(x_ref[...], win_ref[...],
                         preferred_element_type=jnp.float32)


def _interaction_kernel(fs_ref, fp_ref, r_ref, m_ref, ynb_ref,
                        wfs_ref, wfp_ref, wout_ref, bout_ref, wd_ref, bd_ref,
                        out_ref, *, T, N):
    F = wfs_ref.shape[1]
    w = jnp.dot(fs_ref[0], wfs_ref[...], preferred_element_type=jnp.float32)
    w = w + jnp.dot(fp_ref[0], wfp_ref[...],
                    preferred_element_type=jnp.float32)
    r = r_ref[0]
    c = 0.5 * (jnp.cos(r * (jnp.pi / _CUTOFF)) + 1.0)
    c = c * (r < _CUTOFF).astype(jnp.float32) * m_ref[0]
    w = w.reshape(T, N, F) * c[:, :, None]
    y = ynb_ref[...].astype(jnp.float32).reshape(T, N, F)
    agg = jnp.sum(w * y, axis=1)
    v = jnp.dot(agg, wout_ref[...], preferred_element_type=jnp.float32)
    v = jax.nn.softplus(v + bout_ref[...]) - _LOG2
    out_ref[...] = jnp.dot(v, wd_ref[...],
                           preferred_element_type=jnp.float32) + bd_ref[...]


def kernel(x, r_ij, neighbors, neighbor_mask, fsblock_ij, fpblock_ij,
           Wfs, Wfp, Win, Wout, bout, Wd, bd):
    B, A, D = x.shape
    N = neighbors.shape[-1]
    S, F = Wfs.shape

    xf = x.reshape(B * A, D)
    Tk = _pick_tile(B * A, 2048)
    y = pl.pallas_call(
        _in2f_kernel,
        grid=(B * A // Tk,),
        in_specs=[
            pl.BlockSpec((Tk, D), lambda i: (i, 0)),
            pl.BlockSpec((D, F), lambda i: (0, 0)),
        ],
        out_specs=pl.BlockSpec((Tk, F), lambda i: (i, 0)),
        out_shape=jax.ShapeDtypeStruct((B * A, F), jnp.float32),
    )(xf, Win)

    fs_c = fsblock_ij.astype(jnp.bfloat16).reshape(B, A * N, S)
    fp_c = fpblock_ij.astype(jnp.bfloat16).reshape(B, A * N, S)
    wfs_c = Wfs.astype(jnp.bfloat16)
    wfp_c = Wfp.astype(jnp.bfloat16)

    idx = (neighbors.reshape(B, A * N)
           + (jnp.arange(B, dtype=jnp.int32) * A)[:, None]).reshape(B * A * N)
    y_nbh = _sc_gather(y, idx)

    T = _pick_tile(A, 256)
    nt = A // T
    spec_edge = pl.BlockSpec((1, T * N, S), lambda b, i: (b, i, 0))
    spec_ynb = pl.BlockSpec((T * N, F), lambda b, i: (b * nt + i, 0))
    spec_an = pl.BlockSpec((1, T, N), lambda b, i: (b, i, 0))
    full = lambda shape: pl.BlockSpec(shape, lambda b, i: (0,) * len(shape))
    out = pl.pallas_call(
        functools.partial(_interaction_kernel, T=T, N=N),
        grid=(B, nt),
        in_specs=[
            spec_edge, spec_edge, spec_an, spec_an, spec_ynb,
            full((S, F)), full((S, F)), full((F, D)), full((1, D)),
            full((D, D)), full((1, D)),
        ],
        out_specs=pl.BlockSpec((T, D), lambda b, i: (b * nt + i, 0)),
        out_shape=jax.ShapeDtypeStruct((B * A, D), jnp.float32),
    )(fs_c, fp_c,
      r_ij, neighbor_mask, y_nbh,
      wfs_c, wfp_c, Wout, bout.reshape(1, D), Wd, bd.reshape(1, D))
    return out.reshape(B, A, D)


# speedup vs baseline: 1.1355x; 1.0009x over previous
import functools

import jax
import jax.numpy as jnp
from jax import lax
from jax.experimental import pallas as pl
from jax.experimental.pallas import tpu as pltpu
from jax.experimental.pallas import tpu_sc as plsc

_CUTOFF = 5.0
_LOG2 = 0.6931471805599453


def _sc_gather(table, idx):
    M, F = table.shape
    (R,) = idx.shape
    info = plsc.get_sparse_core_info()
    nw = info.num_cores * info.num_subcores
    per_w = R // nw
    RING = 10
    LOOK = 7
    CH = 8
    for c in range(128, 7, -8):
        if per_w % (c * RING) == 0:
            CH = c
            break
    assert per_w * nw == R and per_w % (CH * RING) == 0
    n_chunks = per_w // CH
    n_pass = n_chunks // RING
    mesh = plsc.VectorSubcoreMesh(core_axis_name="c", subcore_axis_name="s")

    @functools.partial(
        pl.kernel,
        mesh=mesh,
        out_type=jax.ShapeDtypeStruct((R, F), table.dtype),
        scratch_types=(
            [pltpu.VMEM((per_w,), jnp.int32)]
            + [pltpu.VMEM((CH, F), table.dtype)] * RING
            + [pltpu.SemaphoreType.DMA] * (2 * RING)
        ),
    )
    def k(table_hbm, idx_hbm, out_hbm, idx_v, *rest):
        bufs = rest[:RING]
        semg = rest[RING:2 * RING]
        semw = rest[2 * RING:]
        wid = lax.axis_index("s") * info.num_cores + lax.axis_index("c")
        base = wid * per_w
        pltpu.sync_copy(idx_hbm.at[pl.ds(base, per_w)], idx_v)

        def g_start(p, b):
            pltpu.make_async_copy(
                table_hbm.at[idx_v.at[pl.ds(p * CH, CH)]], bufs[b], semg[b]
            ).start()

        def g_wait(b):
            pltpu.make_async_copy(
                table_hbm.at[idx_v.at[pl.ds(0, CH)]], bufs[b], semg[b]
            ).wait()

        def w_start(p, b):
            pltpu.make_async_copy(
                bufs[b], out_hbm.at[pl.ds(base + p * CH, CH)], semw[b]
            ).start()

        def w_wait(b):
            pltpu.make_async_copy(
                bufs[b], out_hbm.at[pl.ds(base, CH)], semw[b]
            ).wait()

        def slot(p, b, wait_w, issue):
            g_wait(b)
            w_start(p, b)
            if issue:
                c = (b + LOOK) % RING
                if wait_w:
                    w_wait(c)
                g_start(p + LOOK, c)

        for b in range(LOOK):
            g_start(b, b)
        for b in range(RING):
            slot(b, b, wait_w=(b >= RING - LOOK), issue=True)

        def body(t, carry):
            p0 = t * RING
            for b in range(RING):
                slot(p0 + b, b, wait_w=True, issue=True)
            return carry

        lax.fori_loop(1, n_pass - 1, body, 0)

        p0 = (n_pass - 1) * RING
        for b in range(RING):
            slot(p0 + b, b, wait_w=True, issue=(p0 + b + LOOK < n_chunks))
        for b in range(RING):
            w_wait(b)

    return k(table, idx)


def _pick_tile(n, cap):
    best = 1
    best8 = 0
    for t in range(1, cap + 1):
        if n % t == 0:
            best = t
            if t % 8 == 0:
                best8 = t
    return best8 if best8 else best


def _in2f_kernel(x_ref, win_ref, y_ref):
    y_ref[...] = jnp.dot(x_ref[...], win_ref[...],
                         preferred_element_type=jnp.float32)


def _interaction_kernel(fs_ref, fp_ref, r_ref, m_ref, ynb_ref,
                        wfs_ref, wfp_ref, wout_ref, bout_ref, wd_ref, bd_ref,
                        out_ref, *, T, N):
    F = wfs_ref.shape[1]
    w = jnp.dot(fs_ref[0], wfs_ref[...], preferred_element_type=jnp.float32)
    w = w + jnp.dot(fp_ref[0], wfp_ref[...],
                    preferred_element_type=jnp.float32)
    r = r_ref[0]
    c = 0.5 * (jnp.cos(r * (jnp.pi / _CUTOFF)) + 1.0)
    c = c * (r < _CUTOFF).astype(jnp.float32) * m_ref[0]
    w = w.reshape(T, N, F) * c[:, :, None]
    y = ynb_ref[...].astype(jnp.float32).reshape(T, N, F)
    agg = jnp.sum(w * y, axis=1)
    v = jnp.dot(agg, wout_ref[...], preferred_element_type=jnp.float32)
    v = jax.nn.softplus(v + bout_ref[...]) - _LOG2
    out_ref[...] = jnp.dot(v, wd_ref[...],
                           preferred_element_type=jnp.float32) + bd_ref[...]


def kernel(x, r_ij, neighbors, neighbor_mask, fsblock_ij, fpblock_ij,
           Wfs, Wfp, Win, Wout, bout, Wd, bd):
    B, A, D = x.shape
    N = neighbors.shape[-1]
    S, F = Wfs.shape

    xf = x.reshape(B * A, D)
    Tk = _pick_tile(B * A, 2048)
    y = pl.pallas_call(
        _in2f_kernel,
        grid=(B * A // Tk,),
        in_specs=[
            pl.BlockSpec((Tk, D), lambda i: (i, 0)),
            pl.BlockSpec((D, F), lambda i: (0, 0)),
        ],
        out_specs=pl.BlockSpec((Tk, F), lambda i: (i, 0)),
        out_shape=jax.ShapeDtypeStruct((B * A, F), jnp.float32),
    )(xf, Win)

    fs_c = fsblock_ij.astype(jnp.bfloat16).reshape(B, A * N, S)
    fp_c = fpblock_ij.astype(jnp.bfloat16).reshape(B, A * N, S)
    wfs_c = Wfs.astype(jnp.bfloat16)
    wfp_c = Wfp.astype(jnp.bfloat16)

    idx = (neighbors.reshape(B, A * N)
           + (jnp.arange(B, dtype=jnp.int32) * A)[:, None]).reshape(B * A * N)
    y_nbh = _sc_gather(y, idx)

    T = _pick_tile(A, 256)
    nt = A // T
    spec_edge = pl.BlockSpec((1, T * N, S), lambda b, i: (b, i, 0))
    spec_ynb = pl.BlockSpec((T * N, F), lambda b, i: (b * nt + i, 0))
    spec_an = pl.BlockSpec((1, T, N), lambda b, i: (b, i, 0))
    full = lambda shape: pl.BlockSpec(shape, lambda b, i: (0,) * len(shape))
    out = pl.pallas_call(
        functools.partial(_interaction_kernel, T=T, N=N),
        grid=(B, nt),
        in_specs=[
            spec_edge, spec_edge, spec_an, spec_an, spec_ynb,
            full((S, F)), full((S, F)), full((F, D)), full((1, D)),
            full((D, D)), full((1, D)),
        ],
        out_specs=pl.BlockSpec((T, D), lambda b, i: (b * nt + i, 0)),
        out_shape=jax.ShapeDtypeStruct((B * A, D), jnp.float32),
    )(fs_c, fp_c,
      r_ij, neighbor_mask, y_nbh,
      wfs_c, wfp_c, Wout, bout.reshape(1, D), Wd, bd.reshape(1, D))
    return out.reshape(B, A, D)
